# baseline (speedup 1.0000x reference)
import functools

import jax
import jax.numpy as jnp
from jax import lax
from jax.experimental import pallas as pl
from jax.experimental.pallas import tpu as pltpu
from jax.experimental.pallas import tpu_sc as plsc

_VOCAB = 100000
_EMB = 32
_HID = 128
_CTX = 3
_BATCH = 1024
_NT = 4096
_NE = 2048
_NEH = _NE // 2
_EGRID = (_VOCAB + _NE - 1) // _NE
_HROWS = _EGRID * _NEH
_SHB = _NE.bit_length() - 1
_SHH = _NEH.bit_length() - 1


def _ew_body(embt_ref, w1_ref, out_ref):
    v = embt_ref[...].astype(jnp.bfloat16)
    parts = []
    for c in range(_CTX):
        w1c = w1_ref[pl.ds(c * _EMB, _EMB), :].astype(jnp.bfloat16)
        acc = lax.dot_general(v, w1c, (((0,), (0,)), ((), ())),
                              preferred_element_type=jnp.float32)
        u = lax.bitcast_convert_type(acc, jnp.uint32)
        lo = (u[:_NEH, :] + jnp.uint32(0x8000)) >> 16
        hi = u[_NEH:, :] & jnp.uint32(0xFFFF0000)
        parts.append(lax.bitcast_convert_type(lo | hi, jnp.int32)[None])
    out_ref[...] = jnp.concatenate(parts, axis=0)


def _ew_tc(embt, w1):
    return pl.pallas_call(
        _ew_body,
        grid=(_EGRID,),
        in_specs=[
            pl.BlockSpec((_EMB, _NE), lambda i: (0, i)),
            pl.BlockSpec((_CTX * _EMB, _HID), lambda i: (0, 0)),
        ],
        out_specs=pl.BlockSpec((_CTX, _NEH, _HID), lambda i: (0, i, 0)),
        out_shape=jax.ShapeDtypeStruct((_CTX, _HROWS, _HID), jnp.int32),
    )(embt, w1)


def _gather_sc(ews, idx_flat):
    info = plsc.get_sparse_core_info()
    nc, ns = info.num_cores, info.num_subcores
    nw = nc * ns
    n = idx_flat.shape[0]
    per = n // nw
    mesh = plsc.VectorSubcoreMesh(core_axis_name="c", subcore_axis_name="s")

    @functools.partial(
        pl.kernel,
        mesh=mesh,
        out_type=jax.ShapeDtypeStruct((_CTX * _BATCH, _HID), jnp.int32),
        scratch_types=[
            pltpu.VMEM((per,), jnp.int32),
            pltpu.VMEM((per,), jnp.int32),
            pltpu.VMEM((per, _HID), jnp.int32),
            pltpu.SemaphoreType.DMA,
        ],
    )
    def gather_k(tab_hbm, idx_hbm, out_hbm, idx_v, row_v, rows_v, sem):
        wid = lax.axis_index("s") * nc + lax.axis_index("c")
        base = wid * per
        pltpu.sync_copy(idx_hbm.at[pl.ds(base, per)], idx_v)
        for k in range(per // 16):
            sl = pl.ds(k * 16, 16)
            r = lax.iota(jnp.int32, 16) + (base + 16 * k)
            cvec = lax.shift_right_logical(r, 10)
            xi = idx_v[sl]
            blk = lax.shift_left(lax.shift_right_logical(xi, _SHB), _SHH)
            row_v[sl] = blk + (xi & (_NEH - 1)) + cvec * _HROWS
        pltpu.async_copy(tab_hbm.at[row_v], rows_v, sem).wait()
        pltpu.sync_copy(rows_v, out_hbm.at[pl.ds(base, per)])

    return gather_k(ews, idx_flat)


def _mlp_body(x_ref, g_ref, b1_ref, w2t_ref, b2_ref, out_ref, hidt_ref):
    @pl.when(pl.program_id(0) == 0)
    def _():
        h = jnp.zeros((_BATCH, _HID), jnp.float32)
        for c in range(_CTX):
            u = lax.bitcast_convert_type(g_ref[c], jnp.uint32)
            par = ((x_ref[:, c:c + 1] >> _SHH) & 1)
            parb = jnp.broadcast_to(par, (_BATCH, _HID))
            bits = jnp.where(parb == 1, u & jnp.uint32(0xFFFF0000), u << 16)
            h = h + lax.bitcast_convert_type(bits, jnp.float32)
        h = jnp.maximum(h + b1_ref[...], 0.0)
        hidt_ref[...] = jnp.transpose(h).astype(jnp.bfloat16)

    acc = jnp.dot(w2t_ref[...].astype(jnp.bfloat16), hidt_ref[...],
                  preferred_element_type=jnp.float32)
    out_ref[...] = acc + jnp.transpose(b2_ref[...])


def _mlp_tc(x, g, b1, w2t, b2):
    grid = pl.cdiv(_VOCAB, _NT)
    return pl.pallas_call(
        _mlp_body,
        grid=(grid,),
        in_specs=[
            pl.BlockSpec((_BATCH, _CTX), lambda i: (0, 0)),
            pl.BlockSpec((_CTX, _BATCH, _HID), lambda i: (0, 0, 0)),
            pl.BlockSpec((1, _HID), lambda i: (0, 0)),
            pl.BlockSpec((_NT, _HID), lambda i: (i, 0)),
            pl.BlockSpec((1, _NT), lambda i: (0, i)),
        ],
        out_specs=pl.BlockSpec((_NT, _BATCH), lambda i: (i, 0)),
        out_shape=jax.ShapeDtypeStruct((_VOCAB, _BATCH), jnp.float32),
        scratch_shapes=[pltpu.VMEM((_HID, _BATCH), jnp.bfloat16)],
        compiler_params=pltpu.CompilerParams(
            dimension_semantics=("arbitrary",),
        ),
    )(x, g, b1, w2t, b2)


def kernel(x, emb, W1, b1, W2, b2):
    x = x.astype(jnp.int32)
    idx = x.T.reshape(-1)
    ew = _ew_tc(emb.T, W1)
    ews = ew.reshape(_CTX * _HROWS, _HID)
    g = _gather_sc(ews, idx).reshape(_CTX, _BATCH, _HID)
    out_t = _mlp_tc(x, g, b1.reshape(1, -1), W2.T, b2.reshape(1, -1))
    return out_t.T


# speedup vs baseline: 1.0966x; 1.0966x over previous
import functools

import jax
import jax.numpy as jnp
from jax import lax
from jax.experimental import pallas as pl
from jax.experimental.pallas import tpu as pltpu
from jax.experimental.pallas import tpu_sc as plsc

_VOCAB = 100000
_EMB = 32
_HID = 128
_CTX = 3
_BATCH = 1024
_NT = 4096
_NE = 8192
_NEH = _NE // 2
_EGRID = (_VOCAB + _NE - 1) // _NE
_HROWS = _EGRID * _NEH
_SHB = _NE.bit_length() - 1
_SHH = _NEH.bit_length() - 1


def _ew_body(embt_ref, w1_ref, out_ref):
    v = embt_ref[...].astype(jnp.bfloat16)
    parts = []
    for c in range(_CTX):
        w1c = w1_ref[pl.ds(c * _EMB, _EMB), :].astype(jnp.bfloat16)
        acc = lax.dot_general(v, w1c, (((0,), (0,)), ((), ())),
                              preferred_element_type=jnp.float32)
        u = lax.bitcast_convert_type(acc, jnp.uint32)
        lo = (u[:_NEH, :] + jnp.uint32(0x8000)) >> 16
        hi = u[_NEH:, :] & jnp.uint32(0xFFFF0000)
        parts.append(lax.bitcast_convert_type(lo | hi, jnp.int32)[None])
    out_ref[...] = jnp.concatenate(parts, axis=0)


def _ew_tc(embt, w1):
    return pl.pallas_call(
        _ew_body,
        grid=(_EGRID,),
        in_specs=[
            pl.BlockSpec((_EMB, _NE), lambda i: (0, i)),
            pl.BlockSpec((_CTX * _EMB, _HID), lambda i: (0, 0)),
        ],
        out_specs=pl.BlockSpec((_CTX, _NEH, _HID), lambda i: (0, i, 0)),
        out_shape=jax.ShapeDtypeStruct((_CTX, _HROWS, _HID), jnp.int32),
    )(embt, w1)


def _gather_sc(ews, idx_flat):
    info = plsc.get_sparse_core_info()
    nc, ns = info.num_cores, info.num_subcores
    nw = nc * ns
    n = idx_flat.shape[0]
    per = n // nw
    mesh = plsc.VectorSubcoreMesh(core_axis_name="c", subcore_axis_name="s")

    @functools.partial(
        pl.kernel,
        mesh=mesh,
        out_type=jax.ShapeDtypeStruct((_CTX * _BATCH, _HID), jnp.int32),
        scratch_types=[
            pltpu.VMEM((per,), jnp.int32),
            pltpu.VMEM((per,), jnp.int32),
            pltpu.VMEM((per, _HID), jnp.int32),
            pltpu.SemaphoreType.DMA,
        ],
    )
    def gather_k(tab_hbm, idx_hbm, out_hbm, idx_v, row_v, rows_v, sem):
        wid = lax.axis_index("s") * nc + lax.axis_index("c")
        base = wid * per
        pltpu.sync_copy(idx_hbm.at[pl.ds(base, per)], idx_v)
        for k in range(per // 16):
            sl = pl.ds(k * 16, 16)
            r = lax.iota(jnp.int32, 16) + (base + 16 * k)
            cvec = lax.shift_right_logical(r, 10)
            xi = idx_v[sl]
            blk = lax.shift_left(lax.shift_right_logical(xi, _SHB), _SHH)
            row_v[sl] = blk + (xi & (_NEH - 1)) + cvec * _HROWS
        pltpu.async_copy(tab_hbm.at[row_v], rows_v, sem).wait()
        pltpu.sync_copy(rows_v, out_hbm.at[pl.ds(base, per)])

    return gather_k(ews, idx_flat)


def _mlp_body(x_ref, g_ref, b1_ref, w2t_ref, b2_ref, out_ref, hidt_ref):
    @pl.when(pl.program_id(0) == 0)
    def _():
        h = jnp.zeros((_BATCH, _HID), jnp.float32)
        for c in range(_CTX):
            u = lax.bitcast_convert_type(g_ref[c], jnp.uint32)
            par = ((x_ref[:, c:c + 1] >> _SHH) & 1)
            parb = jnp.broadcast_to(par, (_BATCH, _HID))
            bits = jnp.where(parb == 1, u & jnp.uint32(0xFFFF0000), u << 16)
            h = h + lax.bitcast_convert_type(bits, jnp.float32)
        h = jnp.maximum(h + b1_ref[...], 0.0)
        hidt_ref[...] = jnp.transpose(h).astype(jnp.bfloat16)

    acc = jnp.dot(w2t_ref[...].astype(jnp.bfloat16), hidt_ref[...],
                  preferred_element_type=jnp.float32)
    out_ref[...] = acc + jnp.transpose(b2_ref[...])


def _mlp_tc(x, g, b1, w2t, b2):
    grid = pl.cdiv(_VOCAB, _NT)
    return pl.pallas_call(
        _mlp_body,
        grid=(grid,),
        in_specs=[
            pl.BlockSpec((_BATCH, _CTX), lambda i: (0, 0)),
            pl.BlockSpec((_CTX, _BATCH, _HID), lambda i: (0, 0, 0)),
            pl.BlockSpec((1, _HID), lambda i: (0, 0)),
            pl.BlockSpec((_NT, _HID), lambda i: (i, 0)),
            pl.BlockSpec((1, _NT), lambda i: (0, i)),
        ],
        out_specs=pl.BlockSpec((_NT, _BATCH), lambda i: (i, 0)),
        out_shape=jax.ShapeDtypeStruct((_VOCAB, _BATCH), jnp.float32),
        scratch_shapes=[pltpu.VMEM((_HID, _BATCH), jnp.bfloat16)],
        compiler_params=pltpu.CompilerParams(
            dimension_semantics=("arbitrary",),
        ),
    )(x, g, b1, w2t, b2)


def kernel(x, emb, W1, b1, W2, b2):
    x = x.astype(jnp.int32)
    idx = x.T.reshape(-1)
    ew = _ew_tc(emb.T, W1)
    ews = ew.reshape(_CTX * _HROWS, _HID)
    g = _gather_sc(ews, idx).reshape(_CTX, _BATCH, _HID)
    out_t = _mlp_tc(x, g, b1.reshape(1, -1), W2.T, b2.reshape(1, -1))
    return out_t.T
